# Initial kernel scaffold; baseline (speedup 1.0000x reference)
import jax
import jax.numpy as jnp
from jax.experimental import pallas as pl


def kernel(u, i, U_emb, V_emb):
    raise NotImplementedError("write your pallas kernel here")


# speedup vs baseline: 1.3157x; 1.3157x over previous
import functools

import jax
import jax.numpy as jnp
from jax import lax
from jax.experimental import pallas as pl
from jax.experimental.pallas import tpu as pltpu
from jax.experimental.pallas import tpu_sc as plsc

DIM = 128
LANES = 16
CHUNK = 128


def kernel(u, i, U_emb, V_emb):
    B = u.shape[0]
    info = plsc.get_sparse_core_info()
    nw = info.num_cores * info.num_subcores
    b_per_w = B // nw
    n_chunks = b_per_w // CHUNK

    mesh = plsc.VectorSubcoreMesh(core_axis_name="c", subcore_axis_name="s")

    @functools.partial(
        pl.kernel,
        out_type=jax.ShapeDtypeStruct((B,), jnp.float32),
        mesh=mesh,
        compiler_params=pltpu.CompilerParams(needs_layout_passes=False),
        scratch_types=[
            pltpu.VMEM((CHUNK,), jnp.int32),
            pltpu.VMEM((CHUNK,), jnp.int32),
            pltpu.VMEM((CHUNK, DIM), jnp.float32),
            pltpu.VMEM((CHUNK, DIM), jnp.float32),
            pltpu.VMEM((b_per_w,), jnp.float32),
            pltpu.SemaphoreType.DMA,
            pltpu.SemaphoreType.DMA,
        ],
    )
    def mf(u_hbm, i_hbm, U_hbm, V_hbm, out_hbm,
           uidx_v, iidx_v, urows_v, vrows_v, out_v, sem_u, sem_v):
        wid = lax.axis_index("s") * info.num_cores + lax.axis_index("c")
        wbase = wid * b_per_w
        lane_iota = jax.lax.iota(jnp.int32, LANES)

        def chunk_body(c, carry):
            base = wbase + c * CHUNK
            pltpu.sync_copy(u_hbm.at[pl.ds(base, CHUNK)], uidx_v)
            pltpu.sync_copy(i_hbm.at[pl.ds(base, CHUNK)], iidx_v)
            cu = pltpu.async_copy(U_hbm.at[uidx_v], urows_v, sem_u)
            cv = pltpu.async_copy(V_hbm.at[iidx_v], vrows_v, sem_v)
            cu.wait()
            cv.wait()

            def group_body(g, carry2):
                def row_body(k, tot):
                    r = g * LANES + k
                    acc = urows_v[r, pl.ds(0, LANES)] * vrows_v[r, pl.ds(0, LANES)]
                    for cc in range(1, DIM // LANES):
                        acc = acc + (urows_v[r, pl.ds(cc * LANES, LANES)]
                                     * vrows_v[r, pl.ds(cc * LANES, LANES)])
                    return jnp.where(lane_iota == k, jnp.sum(acc), tot)

                tot = lax.fori_loop(0, LANES, row_body, jnp.zeros((LANES,), jnp.float32))
                out_v[pl.ds(c * CHUNK + g * LANES, LANES)] = tot
                return carry2

            lax.fori_loop(0, CHUNK // LANES, group_body, 0)
            return carry

        lax.fori_loop(0, n_chunks, chunk_body, 0)
        pltpu.sync_copy(out_v, out_hbm.at[pl.ds(wbase, b_per_w)])

    return mf(u.astype(jnp.int32), i.astype(jnp.int32), U_emb, V_emb)


# speedup vs baseline: 1.5427x; 1.1725x over previous
import functools

import jax
import jax.numpy as jnp
from jax import lax
from jax.experimental import pallas as pl
from jax.experimental.pallas import tpu as pltpu
from jax.experimental.pallas import tpu_sc as plsc

DIM = 128
LANES = 16
CHUNK = 128
NBUF = 2


def kernel(u, i, U_emb, V_emb):
    B = u.shape[0]
    info = plsc.get_sparse_core_info()
    nw = info.num_cores * info.num_subcores
    b_per_w = B // nw
    n_chunks = b_per_w // CHUNK

    mesh = plsc.VectorSubcoreMesh(core_axis_name="c", subcore_axis_name="s")

    @functools.partial(
        pl.kernel,
        out_type=jax.ShapeDtypeStruct((B,), jnp.float32),
        mesh=mesh,
        compiler_params=pltpu.CompilerParams(needs_layout_passes=False),
        scratch_types=[
            pltpu.VMEM((b_per_w,), jnp.int32),
            pltpu.VMEM((b_per_w,), jnp.int32),
            pltpu.VMEM((NBUF, CHUNK, DIM), jnp.float32),
            pltpu.VMEM((NBUF, CHUNK, DIM), jnp.float32),
            pltpu.VMEM((b_per_w,), jnp.float32),
            pltpu.SemaphoreType.DMA((NBUF,)),
            pltpu.SemaphoreType.DMA((NBUF,)),
        ],
    )
    def mf(u_hbm, i_hbm, U_hbm, V_hbm, out_hbm,
           uidx_v, iidx_v, urows_v, vrows_v, out_v, sem_u, sem_v):
        wid = lax.axis_index("s") * info.num_cores + lax.axis_index("c")
        wbase = wid * b_per_w
        lane_iota = jax.lax.iota(jnp.int32, LANES)

        pltpu.sync_copy(u_hbm.at[pl.ds(wbase, b_per_w)], uidx_v)
        pltpu.sync_copy(i_hbm.at[pl.ds(wbase, b_per_w)], iidx_v)

        def start(c, slot):
            return (
                pltpu.async_copy(U_hbm.at[uidx_v.at[pl.ds(c * CHUNK, CHUNK)]],
                                 urows_v.at[slot], sem_u.at[slot]),
                pltpu.async_copy(V_hbm.at[iidx_v.at[pl.ds(c * CHUNK, CHUNK)]],
                                 vrows_v.at[slot], sem_v.at[slot]),
            )

        def compute(c, slot):
            ur = urows_v.at[slot]
            vr = vrows_v.at[slot]

            def group_body(g, carry2):
                def row_body(k, tot):
                    r = g * LANES + k
                    acc = ur[r, pl.ds(0, LANES)] * vr[r, pl.ds(0, LANES)]
                    for cc in range(1, DIM // LANES):
                        acc = acc + (ur[r, pl.ds(cc * LANES, LANES)]
                                     * vr[r, pl.ds(cc * LANES, LANES)])
                    return jnp.where(lane_iota == k, jnp.sum(acc), tot)

                tot = lax.fori_loop(0, LANES, row_body,
                                    jnp.zeros((LANES,), jnp.float32),
                                    unroll=4)
                out_v[pl.ds(c * CHUNK + g * LANES, LANES)] = tot
                return carry2

            lax.fori_loop(0, CHUNK // LANES, group_body, 0)

        copies = {0: start(0, 0)}
        for c in range(n_chunks):
            if c + 1 < n_chunks:
                copies[c + 1] = start(c + 1, (c + 1) % NBUF)
            cu, cv = copies.pop(c)
            cu.wait()
            cv.wait()
            compute(c, c % NBUF)

        pltpu.sync_copy(out_v, out_hbm.at[pl.ds(wbase, b_per_w)])

    return mf(u.astype(jnp.int32), i.astype(jnp.int32), U_emb, V_emb)
